# baseline (speedup 1.0000x reference)
import functools

import jax
import jax.numpy as jnp
from jax import lax
from jax.experimental import pallas as pl
from jax.experimental.pallas import tpu as pltpu
from jax.experimental.pallas import tpu_sc as plsc

VOCAB = 49408
D = 512
S = 77
B = 4096
NC = 2
NS = 16
NW = NC * NS
STRIPE = B // NW
WB = 64
NWIN = S * 2
LANES = 16


def kernel(input_ids, token_embedding, position_embedding):
    ids_t = input_ids.astype(jnp.int32).T
    mesh = plsc.VectorSubcoreMesh(core_axis_name="c", subcore_axis_name="s")

    @functools.partial(
        pl.kernel,
        out_type=jax.ShapeDtypeStruct((B, S * D), jnp.float32),
        mesh=mesh,
        scratch_types=[
            pltpu.VMEM((S, STRIPE), jnp.int32),
            pltpu.VMEM((S, D), jnp.float32),
            pltpu.VMEM((WB, D), jnp.float32),
            pltpu.VMEM((WB, D), jnp.float32),
            pltpu.SemaphoreType.DMA,
            pltpu.SemaphoreType.DMA,
            pltpu.SemaphoreType.DMA,
            pltpu.SemaphoreType.DMA,
        ],
    )
    def gather_add(ids_hbm, tab_hbm, pos_hbm, out_hbm,
                   idx_v, pos_v, rows0_v, rows1_v,
                   gsem0, gsem1, osem0, osem1):
        wid = lax.axis_index("s") * NC + lax.axis_index("c")
        b0 = wid * STRIPE
        rows = (rows0_v, rows1_v)
        gsem = (gsem0, gsem1)
        osem = (osem0, osem1)

        pltpu.sync_copy(ids_hbm.at[:, pl.ds(b0, STRIPE)], idx_v)
        pltpu.sync_copy(pos_hbm, pos_v)

        def idx_slice(s, h):
            return idx_v.at[s, pl.ds(h * WB, WB)]

        def out_slice(s, h):
            return out_hbm.at[pl.ds(b0 + h * WB, WB), pl.ds(s * D, D)]

        pltpu.async_copy(tab_hbm.at[idx_slice(0, 0)], rows0_v, gsem0)

        @pl.loop(0, NWIN, step=2)
        def _(w):
            for boff in range(2):
                ww = w + boff
                bsel = boff
                s = ww >> 1
                h = ww & 1

                @pl.when(ww > 0)
                def _():
                    pltpu.make_async_copy(
                        rows[1 - bsel], out_slice(s, h), osem[1 - bsel]
                    ).wait()

                nxt = ww + 1

                @pl.when(nxt < NWIN)
                def _():
                    pltpu.async_copy(
                        tab_hbm.at[idx_slice(nxt >> 1, nxt & 1)],
                        rows[1 - bsel],
                        gsem[1 - bsel],
                    )

                pltpu.make_async_copy(
                    tab_hbm.at[idx_slice(s, h)], rows[bsel], gsem[bsel]
                ).wait()
                for c in range(D // LANES):
                    pc = pos_v[s, pl.ds(c * LANES, LANES)]

                    @pl.loop(0, WB, unroll=8)
                    def _(r):
                        plsc.addupdate(rows[bsel].at[r, pl.ds(c * LANES, LANES)], pc)

                pltpu.async_copy(rows[bsel], out_slice(s, h), osem[bsel])

        pltpu.make_async_copy(rows1_v, out_slice(S - 1, 1), osem1).wait()

    out = gather_add(ids_t, token_embedding, position_embedding)
    one = jnp.float32(1.0) + jnp.float32(0.0) * position_embedding[0, 0]
    return out.reshape(B, S, D) * one


# speedup vs baseline: 1.4555x; 1.4555x over previous
import functools

import jax
import jax.numpy as jnp
from jax import lax
from jax.experimental import pallas as pl
from jax.experimental.pallas import tpu as pltpu
from jax.experimental.pallas import tpu_sc as plsc

VOCAB = 49408
D = 512
S = 77
B = 4096
NC = 2
NS = 16
NW = NC * NS
STRIPE = B // NW
WB = 32
NH = STRIPE // WB
NWIN = S * NH
RING = 4
LANES = 16


def kernel(input_ids, token_embedding, position_embedding):
    ids_t = input_ids.astype(jnp.int32).T
    mesh = plsc.VectorSubcoreMesh(core_axis_name="c", subcore_axis_name="s")

    @functools.partial(
        pl.kernel,
        out_type=jax.ShapeDtypeStruct((B, S * D), jnp.float32),
        mesh=mesh,
        scratch_types=[
            pltpu.VMEM((S, STRIPE), jnp.int32),
            pltpu.VMEM((S, D), jnp.float32),
            pltpu.VMEM((WB, D), jnp.float32),
            pltpu.VMEM((WB, D), jnp.float32),
            pltpu.VMEM((WB, D), jnp.float32),
            pltpu.VMEM((WB, D), jnp.float32),
            pltpu.SemaphoreType.DMA,
            pltpu.SemaphoreType.DMA,
            pltpu.SemaphoreType.DMA,
            pltpu.SemaphoreType.DMA,
            pltpu.SemaphoreType.DMA,
            pltpu.SemaphoreType.DMA,
            pltpu.SemaphoreType.DMA,
            pltpu.SemaphoreType.DMA,
        ],
    )
    def gather_add(ids_hbm, tab_hbm, pos_hbm, out_hbm,
                   idx_v, pos_v, r0, r1, r2, r3,
                   g0, g1, g2, g3, o0, o1, o2, o3):
        wid = lax.axis_index("s") * NC + lax.axis_index("c")
        b0 = wid * STRIPE
        rows = (r0, r1, r2, r3)
        gsem = (g0, g1, g2, g3)
        osem = (o0, o1, o2, o3)

        pltpu.sync_copy(ids_hbm.at[:, pl.ds(b0, STRIPE)], idx_v)
        pltpu.sync_copy(pos_hbm, pos_v)

        def idx_slice(w):
            return idx_v.at[w // NH, pl.ds((w % NH) * WB, WB)]

        def out_slice(w):
            return out_hbm.at[
                pl.ds(b0 + (w % NH) * WB, WB), pl.ds((w // NH) * D, D)
            ]

        def gather(w, bsel):
            pltpu.async_copy(tab_hbm.at[idx_slice(w)], rows[bsel], gsem[bsel])

        for k in range(RING - 1):
            gather(k, k)

        @pl.loop(0, NWIN, step=RING)
        def _(w):
            for boff in range(RING):
                ww = w + boff
                bsel = boff
                nsel = (boff + RING - 1) % RING

                @pl.when(ww > 0)
                def _():
                    pltpu.make_async_copy(
                        rows[nsel], out_slice(ww - 1), osem[nsel]
                    ).wait()

                nxt = ww + RING - 1

                @pl.when(nxt < NWIN)
                def _():
                    gather(nxt, nsel)

                pltpu.make_async_copy(
                    tab_hbm.at[idx_slice(ww)], rows[bsel], gsem[bsel]
                ).wait()
                s = ww // NH

                @pl.loop(0, D, step=LANES)
                def _(c0):
                    pc = pos_v[s, pl.ds(c0, LANES)]

                    @pl.loop(0, WB, unroll=8)
                    def _(r):
                        plsc.addupdate(rows[bsel].at[r, pl.ds(c0, LANES)], pc)

                pltpu.async_copy(rows[bsel], out_slice(ww), osem[bsel])

        pltpu.make_async_copy(
            rows[(NWIN - 1) % RING], out_slice(NWIN - 1), osem[(NWIN - 1) % RING]
        ).wait()

    out = gather_add(ids_t, token_embedding, position_embedding)
    return out.reshape(B, S, D)


# speedup vs baseline: 1.4573x; 1.0012x over previous
import functools

import jax
import jax.numpy as jnp
from jax import lax
from jax.experimental import pallas as pl
from jax.experimental.pallas import tpu as pltpu
from jax.experimental.pallas import tpu_sc as plsc

VOCAB = 49408
D = 512
S = 77
B = 4096
NC = 2
NS = 16
NW = NC * NS
STRIPE = B // NW
WB = 32
NH = STRIPE // WB
NWIN = S * NH
RING = 4
LANES = 16


def kernel(input_ids, token_embedding, position_embedding):
    ids_t = input_ids.astype(jnp.int32).T
    mesh = plsc.VectorSubcoreMesh(core_axis_name="c", subcore_axis_name="s")

    @functools.partial(
        pl.kernel,
        out_type=jax.ShapeDtypeStruct((B, S * D), jnp.float32),
        mesh=mesh,
        scratch_types=[
            pltpu.VMEM((S, STRIPE), jnp.int32),
            pltpu.VMEM((S, D), jnp.float32),
            pltpu.VMEM((WB, D), jnp.float32),
            pltpu.VMEM((WB, D), jnp.float32),
            pltpu.VMEM((WB, D), jnp.float32),
            pltpu.VMEM((WB, D), jnp.float32),
            pltpu.SemaphoreType.DMA,
            pltpu.SemaphoreType.DMA,
            pltpu.SemaphoreType.DMA,
            pltpu.SemaphoreType.DMA,
            pltpu.SemaphoreType.DMA,
            pltpu.SemaphoreType.DMA,
            pltpu.SemaphoreType.DMA,
            pltpu.SemaphoreType.DMA,
        ],
    )
    def gather_add(ids_hbm, tab_hbm, pos_hbm, out_hbm,
                   idx_v, pos_v, r0, r1, r2, r3,
                   g0, g1, g2, g3, o0, o1, o2, o3):
        wid = lax.axis_index("s") * NC + lax.axis_index("c")
        b0 = wid * STRIPE
        rows = (r0, r1, r2, r3)
        gsem = (g0, g1, g2, g3)
        osem = (o0, o1, o2, o3)

        pltpu.sync_copy(ids_hbm.at[:, pl.ds(b0, STRIPE)], idx_v)
        pltpu.sync_copy(pos_hbm, pos_v)

        def idx_slice(w):
            return idx_v.at[w // NH, pl.ds((w % NH) * WB, WB)]

        def out_slice(w):
            return out_hbm.at[
                pl.ds(b0 + (w % NH) * WB, WB), pl.ds((w // NH) * D, D)
            ]

        def gather(w, bsel):
            pltpu.async_copy(tab_hbm.at[idx_slice(w)], rows[bsel], gsem[bsel])

        for k in range(RING - 1):
            gather(k, k)

        @pl.loop(0, NWIN, step=RING)
        def _(w):
            for boff in range(RING):
                ww = w + boff
                bsel = boff
                nsel = (boff + RING - 1) % RING

                @pl.when(ww > 0)
                def _():
                    pltpu.make_async_copy(
                        rows[nsel], out_slice(ww - 1), osem[nsel]
                    ).wait()

                nxt = ww + RING - 1

                @pl.when(nxt < NWIN)
                def _():
                    gather(nxt, nsel)

                pltpu.make_async_copy(
                    tab_hbm.at[idx_slice(ww)], rows[bsel], gsem[bsel]
                ).wait()
                s = ww // NH

                @pl.loop(0, D, step=LANES)
                def _(c0):
                    pc = pos_v[s, pl.ds(c0, LANES)]

                    @pl.loop(0, WB, unroll=16)
                    def _(r):
                        plsc.addupdate(rows[bsel].at[r, pl.ds(c0, LANES)], pc)

                pltpu.async_copy(rows[bsel], out_slice(ww), osem[bsel])

        pltpu.make_async_copy(
            rows[(NWIN - 1) % RING], out_slice(NWIN - 1), osem[(NWIN - 1) % RING]
        ).wait()

    out = gather_add(ids_t, token_embedding, position_embedding)
    return out.reshape(B, S, D)
